# baseline (speedup 1.0000x reference)
import functools

import jax
import jax.numpy as jnp
from jax import lax
from jax.experimental import pallas as pl
from jax.experimental.pallas import tpu as pltpu
from jax.experimental.pallas import tpu_sc as plsc

_NB = 4
_C = 400
_TI = 512


def _sc_gather(idx2d, weight, B, D, NC, NS):
    NW = NC * NS
    b_per_w = B // NW
    C = _C
    NB = _NB
    n_chunks = b_per_w // C
    mesh = plsc.VectorSubcoreMesh(core_axis_name="c", subcore_axis_name="s")

    @functools.partial(
        pl.kernel,
        mesh=mesh,
        out_type=jax.ShapeDtypeStruct((B, D), jnp.float32),
        scratch_types=[
            pltpu.VMEM((n_chunks, C), jnp.int32),
            pltpu.VMEM((NB, C, D), jnp.float32),
            pltpu.SemaphoreType.DMA((NB,)),
            pltpu.SemaphoreType.DMA((NB,)),
        ],
        compiler_params=pltpu.CompilerParams(use_tc_tiling_on_sc=False),
    )
    def emb(idx_hbm, table_hbm, out_hbm, idx_v, rows_v, gsem, ssem):
        wid = lax.axis_index("s") * NC + lax.axis_index("c")
        base = wid * b_per_w

        pltpu.sync_copy(idx_hbm.at[pl.ds(wid * n_chunks, n_chunks)], idx_v)

        def start_gather(c, b):
            pltpu.async_copy(table_hbm.at[idx_v.at[c]], rows_v.at[b], gsem.at[b])

        def wait_gather(b):
            pltpu.make_async_copy(
                table_hbm.at[pl.ds(0, C)], rows_v.at[b], gsem.at[b]).wait()

        def start_store(c, b):
            pltpu.async_copy(
                rows_v.at[b], out_hbm.at[pl.ds(base + c * C, C)], ssem.at[b])

        def wait_store(b):
            pltpu.make_async_copy(
                rows_v.at[b], out_hbm.at[pl.ds(base, C)], ssem.at[b]).wait()

        start_gather(0, 0)
        start_gather(1, 1)

        for b in range(NB):
            c = b
            wait_gather(b)
            start_store(c, b)
            if c + 2 < n_chunks:
                if c >= 2:
                    wait_store((b + 2) % NB)
                start_gather(c + 2, (b + 2) % NB)

        def body(k, carry):
            i = k * NB
            for b in range(NB):
                c = i + b
                wait_gather(b)
                start_store(c, b)
                wait_store((b + 2) % NB)
                start_gather(c + 2, (b + 2) % NB)
            return carry

        lax.fori_loop(1, n_chunks // NB - 1, body, 0)

        for b in range(NB):
            c = n_chunks - NB + b
            wait_gather(b)
            start_store(c, b)
            if c + 2 < n_chunks:
                wait_store((b + 2) % NB)
                start_gather(c + 2, (b + 2) % NB)

        for b in range(NB):
            wait_store(b)

    return emb(idx2d, weight)


def _tc_retile(i2, NJ, NI, D):
    nib = NI // _TI
    nq = _TI // 128

    def body(x_ref, o_ref):
        x = x_ref[...]
        for q in range(nq):
            xs = x[q * 128:(q + 1) * 128, :]
            o_ref[0, :, q] = xs.T.reshape(D // 8, 8, 128)

    return pl.pallas_call(
        body,
        grid=(NJ, nib),
        in_specs=[pl.BlockSpec((_TI, D), lambda j, ib: (j * nib + ib, 0))],
        out_specs=pl.BlockSpec(
            (1, D // 8, nq, 8, 128), lambda j, ib: (j, 0, ib, 0, 0)),
        out_shape=jax.ShapeDtypeStruct(
            (NJ, D // 8, NI // 128, 8, 128), jnp.float32),
    )(i2)


def kernel(input, weight):
    S0, S1 = input.shape
    D = weight.shape[1]
    B = S0 * S1
    info = plsc.get_sparse_core_info()
    NW = info.num_cores * info.num_subcores
    n_chunks = (B // NW) // _C
    idx2d = input.T.reshape(NW * n_chunks, _C).astype(jnp.int32)
    i2 = _sc_gather(idx2d, weight, B, D, info.num_cores, info.num_subcores)
    o5 = _tc_retile(i2, S1, S0, D)
    out = jnp.transpose(o5, (2, 4, 0, 1, 3)).reshape(S0, S1, D)
    return out


# speedup vs baseline: 1.6200x; 1.6200x over previous
import functools

import jax
import jax.numpy as jnp
from jax import lax
from jax.experimental import pallas as pl
from jax.experimental.pallas import tpu as pltpu
from jax.experimental.pallas import tpu_sc as plsc

_NB = 4
_C = 400


def _emb_gather(idx2d, weight, B, D, NC, NS):
    NW = NC * NS
    b_per_w = B // NW
    C = _C
    NB = _NB
    n_chunks = b_per_w // C
    mesh = plsc.VectorSubcoreMesh(core_axis_name="c", subcore_axis_name="s")

    @functools.partial(
        pl.kernel,
        mesh=mesh,
        out_type=jax.ShapeDtypeStruct((B, D), jnp.float32),
        scratch_types=[
            pltpu.VMEM((n_chunks, C), jnp.int32),
            pltpu.VMEM((NB, C, D), jnp.float32),
            pltpu.SemaphoreType.DMA((NB,)),
            pltpu.SemaphoreType.DMA((NB,)),
        ],
        compiler_params=pltpu.CompilerParams(use_tc_tiling_on_sc=False),
    )
    def emb(idx_hbm, table_hbm, out_hbm, idx_v, rows_v, gsem, ssem):
        wid = lax.axis_index("s") * NC + lax.axis_index("c")
        base = wid * b_per_w

        pltpu.sync_copy(idx_hbm.at[pl.ds(wid * n_chunks, n_chunks)], idx_v)

        def start_gather(c, b):
            pltpu.async_copy(table_hbm.at[idx_v.at[c]], rows_v.at[b], gsem.at[b])

        def wait_gather(b):
            pltpu.make_async_copy(
                table_hbm.at[pl.ds(0, C)], rows_v.at[b], gsem.at[b]).wait()

        def start_store(c, b):
            pltpu.async_copy(
                rows_v.at[b], out_hbm.at[pl.ds(base + c * C, C)], ssem.at[b])

        def wait_store(b):
            pltpu.make_async_copy(
                rows_v.at[b], out_hbm.at[pl.ds(base, C)], ssem.at[b]).wait()

        start_gather(0, 0)
        start_gather(1, 1)

        for b in range(NB):
            c = b
            wait_gather(b)
            start_store(c, b)
            if c + 2 < n_chunks:
                if c >= 2:
                    wait_store((b + 2) % NB)
                start_gather(c + 2, (b + 2) % NB)

        def body(k, carry):
            i = k * NB
            for b in range(NB):
                c = i + b
                wait_gather(b)
                start_store(c, b)
                wait_store((b + 2) % NB)
                start_gather(c + 2, (b + 2) % NB)
            return carry

        lax.fori_loop(1, n_chunks // NB - 1, body, 0)

        for b in range(NB):
            c = n_chunks - NB + b
            wait_gather(b)
            start_store(c, b)
            if c + 2 < n_chunks:
                wait_store((b + 2) % NB)
                start_gather(c + 2, (b + 2) % NB)

        for b in range(NB):
            wait_store(b)

    return emb(idx2d, weight)


def kernel(input, weight):
    S0, S1 = input.shape
    D = weight.shape[1]
    B = S0 * S1
    info = plsc.get_sparse_core_info()
    NW = info.num_cores * info.num_subcores
    n_chunks = (B // NW) // _C
    idx2d = input.reshape(NW * n_chunks, _C).astype(jnp.int32)
    out = _emb_gather(idx2d, weight, B, D, info.num_cores, info.num_subcores)
    return out.reshape(S0, S1, D)
